# baseline (speedup 1.0000x reference)
import functools

import jax
import jax.numpy as jnp
from jax import lax
from jax.experimental import pallas as pl
from jax.experimental.pallas import tpu as pltpu
from jax.experimental.pallas import tpu_sc as plsc

NN = 10000
EE = 160000
HD = 256
HF = 128
OUTD = 128
NL = 3
NG = 64
NSUB = 16
CHUNK = 80
NCHUNK = EE // (NSUB * CHUNK)
NCOMBO = 125
BN = 2000
NBLK = NN // BN


def _atom_body(x_ref, aemb_ref, out_ref):
    xb = x_ref[...]
    cols = lax.broadcasted_iota(jnp.int32, (BN, 128), 1)
    oh = jnp.concatenate(
        [(xb[:, f:f + 1] == cols).astype(jnp.float32) for f in range(9)],
        axis=1)
    h = jnp.dot(oh, aemb_ref[...], preferred_element_type=jnp.float32)
    out_ref[0] = h[:, :HF]
    out_ref[1] = h[:, HF:]


def _atom_encode(x, aemb):
    return pl.pallas_call(
        _atom_body,
        grid=(NBLK,),
        in_specs=[pl.BlockSpec((BN, 9), lambda i: (i, 0)),
                  pl.BlockSpec((9 * 128, HD), lambda i: (0, 0))],
        out_specs=pl.BlockSpec((2, BN, HF), lambda i: (0, i, 0)),
        out_shape=jax.ShapeDtypeStruct((2, NN, HF), jnp.float32),
    )(x, aemb)


def _etab_body(bemb_ref, out_ref):
    c = lax.broadcasted_iota(jnp.int32, (NCOMBO, 16), 0)
    k = lax.broadcasted_iota(jnp.int32, (NCOMBO, 16), 1)
    m = jnp.concatenate([(c // 25 == k).astype(jnp.float32),
                         ((c // 5) % 5 == k).astype(jnp.float32),
                         (c % 5 == k).astype(jnp.float32)], axis=1)
    for l in range(NL):
        et = jnp.dot(m, bemb_ref[l], preferred_element_type=jnp.float32)
        out_ref[l, 0] = et[:, :HF]
        out_ref[l, 1] = et[:, HF:]


def _etabs(bemb):
    return pl.pallas_call(
        _etab_body,
        grid=(1,),
        in_specs=[pl.BlockSpec((NL, 48, HD), lambda i: (0, 0, 0))],
        out_specs=pl.BlockSpec((NL, 2, NCOMBO, HF), lambda i: (0, 0, 0, 0)),
        out_shape=jax.ShapeDtypeStruct((NL, 2, NCOMBO, HF), jnp.float32),
    )(bemb)


def _combo_body(ea_ref, out_ref):
    out_ref[...] = ea_ref[0] * 25 + ea_ref[1] * 5 + ea_ref[2]


def _combo_ids(ea_t):
    return pl.pallas_call(
        _combo_body,
        grid=(1,),
        in_specs=[pl.BlockSpec((3, 1250, 128), lambda i: (0, 0, 0))],
        out_specs=pl.BlockSpec((1250, 128), lambda i: (0, 0)),
        out_shape=jax.ShapeDtypeStruct((1250, 128), jnp.int32),
    )(ea_t)


def _sc_msgpass(h_split, etab, idx3):
    mesh = plsc.VectorSubcoreMesh(core_axis_name="c", subcore_axis_name="s")

    @functools.partial(
        pl.kernel,
        out_type=jax.ShapeDtypeStruct((2, NN, HF), jnp.float32),
        mesh=mesh,
        scratch_types=[
            pltpu.VMEM((3, 3, CHUNK), jnp.int32),
            pltpu.VMEM((2, CHUNK, HF), jnp.float32),
            pltpu.VMEM((2, CHUNK, HF), jnp.float32),
            pltpu.VMEM((16, HF), jnp.float32),
            pltpu.VMEM_SHARED((NN, HF), jnp.float32),
            pltpu.VMEM_SHARED((NCOMBO, HF), jnp.float32),
            pltpu.SemaphoreType.DMA,
            pltpu.SemaphoreType.DMA,
            pltpu.SemaphoreType.DMA,
            pltpu.SemaphoreType.DMA,
        ],
    )
    def k(h_hbm, etab_hbm, idx_hbm, out_hbm,
          islot, hrows2, erows2, ztile, agg_sh, etab_sh,
          sem_h, sem_e, sem_i, sem_s):
        c = lax.axis_index("c")
        s = lax.axis_index("s")

        @pl.when(s == 0)
        def _():
            pltpu.sync_copy(etab_hbm.at[c], etab_sh)

        def _zrow(r, carry):
            for kk in range(8):
                ztile[r, pl.ds(kk * 16, 16)] = jnp.zeros((16,), jnp.float32)
            return carry
        lax.fori_loop(0, 16, _zrow, 0)
        w0 = s * 624

        def _zcp(t, carry):
            pltpu.sync_copy(ztile, agg_sh.at[pl.ds(w0 + t * 16, 16)])
            return carry
        lax.fori_loop(0, 39, _zcp, 0)

        @pl.when(s == NSUB - 1)
        def _():
            pltpu.sync_copy(ztile, agg_sh.at[pl.ds(9984, 16)])
        plsc.subcore_barrier()

        htab = h_hbm.at[c]
        etv = etab_hbm.at[c]
        irows = idx_hbm.at[s]
        hdummy = htab.at[pl.ds(0, CHUNK)]
        edummy = etv.at[pl.ds(0, CHUNK)]
        idummy = irows.at[0]

        pltpu.sync_copy(irows.at[0], islot.at[0])
        pltpu.async_copy(htab.at[islot.at[0].at[0]], hrows2.at[0], sem_h)
        pltpu.async_copy(etab_sh.at[islot.at[0].at[2]], erows2.at[0], sem_e)
        pltpu.async_copy(irows.at[1], islot.at[1], sem_i)

        def _chunk(j, carry):
            b = jnp.bitwise_and(j, 1)
            nb = 1 - b
            b3 = lax.rem(j, 3)
            nb3 = lax.rem(j + 1, 3)
            pb3 = lax.rem(j + 2, 3)

            @pl.when(j < NCHUNK - 1)
            def _():
                pltpu.make_async_copy(idummy, islot.at[nb3], sem_i).wait()

                @pl.when(j >= 1)
                def _():
                    pltpu.make_async_copy(hdummy, hrows2.at[nb], sem_s).wait()
                pltpu.async_copy(htab.at[islot.at[nb3].at[0]],
                                 hrows2.at[nb], sem_h)
                pltpu.async_copy(etab_sh.at[islot.at[nb3].at[2]],
                                 erows2.at[nb], sem_e)

            pltpu.make_async_copy(hdummy, hrows2.at[b], sem_h).wait()
            pltpu.make_async_copy(edummy, erows2.at[b], sem_e).wait()

            @pl.when(j < NCHUNK - 2)
            def _():
                pltpu.async_copy(irows.at[j + 2], islot.at[pb3], sem_i)

            def _compute(slot):
                @plsc.parallel_loop(0, CHUNK, step=1, unroll=4)
                def _edge(r):
                    for kk in range(8):
                        sl = pl.ds(kk * 16, 16)
                        hrows2[slot, r, sl] = jnp.maximum(
                            hrows2[slot, r, sl] + erows2[slot, r, sl], 0.0)

            @pl.when(b == 0)
            def _():
                _compute(0)

            @pl.when(b == 1)
            def _():
                _compute(1)
            pltpu.async_copy(hrows2.at[b], agg_sh.at[islot.at[b3].at[1]],
                             sem_s, add=True)
            return carry
        lax.fori_loop(0, NCHUNK, _chunk, 0)

        pltpu.make_async_copy(hdummy, hrows2.at[0], sem_s).wait()
        pltpu.make_async_copy(hdummy, hrows2.at[1], sem_s).wait()
        plsc.subcore_barrier()

        w0 = s * 624
        pltpu.sync_copy(agg_sh.at[pl.ds(w0, 624)],
                        out_hbm.at[c].at[pl.ds(w0, 624)])

        @pl.when(s == NSUB - 1)
        def _():
            pltpu.sync_copy(agg_sh.at[pl.ds(9984, 16)],
                            out_hbm.at[c].at[pl.ds(9984, 16)])

    return k(h_split, etab, idx3)


def _mlp1_body(scale_ref, h_ref, agg_ref, w1_ref, b1_ref, y1_ref, st_ref):
    i = pl.program_id(0)
    sc = scale_ref[0, 0]
    z0 = sc * h_ref[0] + agg_ref[0]
    z1 = sc * h_ref[1] + agg_ref[1]
    y = (jnp.dot(z0, w1_ref[0:HF, :], preferred_element_type=jnp.float32)
         + jnp.dot(z1, w1_ref[HF:, :], preferred_element_type=jnp.float32)
         + b1_ref[...])
    y1_ref[...] = y

    @pl.when(i == 0)
    def _():
        st_ref[...] = jnp.zeros((2, 2 * HD), jnp.float32)
    st_ref[...] += jnp.concatenate(
        [jnp.sum(y, axis=0, keepdims=True),
         jnp.sum(y * y, axis=0, keepdims=True)], axis=0)


def _mlp1(scale, h_split, agg_split, w1, b1):
    return pl.pallas_call(
        _mlp1_body,
        grid=(NBLK,),
        in_specs=[pl.BlockSpec(memory_space=pltpu.SMEM),
                  pl.BlockSpec((2, BN, HF), lambda i: (0, i, 0)),
                  pl.BlockSpec((2, BN, HF), lambda i: (0, i, 0)),
                  pl.BlockSpec((HD, 2 * HD), lambda i: (0, 0)),
                  pl.BlockSpec((1, 2 * HD), lambda i: (0, 0))],
        out_specs=[pl.BlockSpec((BN, 2 * HD), lambda i: (i, 0)),
                   pl.BlockSpec((2, 2 * HD), lambda i: (0, 0))],
        out_shape=[jax.ShapeDtypeStruct((NN, 2 * HD), jnp.float32),
                   jax.ShapeDtypeStruct((2, 2 * HD), jnp.float32)],
    )(scale, h_split, agg_split, w1, b1)


def _mlp2_body(y1_ref, st_ref, g_ref, bb_ref, w2_ref, b2_ref, y2_ref, st2_ref):
    i = pl.program_id(0)
    m = st_ref[0:1, :] * (1.0 / NN)
    v = st_ref[1:2, :] * (1.0 / NN) - m * m
    rs = lax.rsqrt(v + 1e-5)
    yn = jnp.maximum((y1_ref[...] - m) * rs * g_ref[...] + bb_ref[...], 0.0)
    y2 = jnp.dot(yn, w2_ref[...], preferred_element_type=jnp.float32) + b2_ref[...]
    y2_ref[...] = y2

    @pl.when(i == 0)
    def _():
        st2_ref[...] = jnp.zeros((2, HD), jnp.float32)
    st2_ref[...] += jnp.concatenate(
        [jnp.sum(y2, axis=0, keepdims=True),
         jnp.sum(y2 * y2, axis=0, keepdims=True)], axis=0)


def _mlp2(y1, st, g, bb, w2, b2):
    return pl.pallas_call(
        _mlp2_body,
        grid=(NBLK,),
        in_specs=[pl.BlockSpec((BN, 2 * HD), lambda i: (i, 0)),
                  pl.BlockSpec((2, 2 * HD), lambda i: (0, 0)),
                  pl.BlockSpec((1, 2 * HD), lambda i: (0, 0)),
                  pl.BlockSpec((1, 2 * HD), lambda i: (0, 0)),
                  pl.BlockSpec((2 * HD, HD), lambda i: (0, 0)),
                  pl.BlockSpec((1, HD), lambda i: (0, 0))],
        out_specs=[pl.BlockSpec((BN, HD), lambda i: (i, 0)),
                   pl.BlockSpec((2, HD), lambda i: (0, 0))],
        out_shape=[jax.ShapeDtypeStruct((NN, HD), jnp.float32),
                   jax.ShapeDtypeStruct((2, HD), jnp.float32)],
    )(y1, st, g, bb, w2, b2)


def _obn_body(y2_ref, st2_ref, g_ref, bb_ref, out_ref):
    m = st2_ref[0:1, :] * (1.0 / NN)
    v = st2_ref[1:2, :] * (1.0 / NN) - m * m
    rs = lax.rsqrt(v + 1e-5)
    yn = jnp.maximum((y2_ref[...] - m) * rs * g_ref[...] + bb_ref[...], 0.0)
    out_ref[0] = yn[:, :HF]
    out_ref[1] = yn[:, HF:]


def _obn(y2, st2, g, bb):
    return pl.pallas_call(
        _obn_body,
        grid=(NBLK,),
        in_specs=[pl.BlockSpec((BN, HD), lambda i: (i, 0)),
                  pl.BlockSpec((2, HD), lambda i: (0, 0)),
                  pl.BlockSpec((1, HD), lambda i: (0, 0)),
                  pl.BlockSpec((1, HD), lambda i: (0, 0))],
        out_specs=pl.BlockSpec((2, BN, HF), lambda i: (0, i, 0)),
        out_shape=jax.ShapeDtypeStruct((2, NN, HF), jnp.float32),
    )(y2, st2, g, bb)


def _pool_body(y2_ref, b_ref, ow_ref, ob_ref, out_ref, sums, cnt):
    i = pl.program_id(0)

    @pl.when(i == 0)
    def _():
        sums[...] = jnp.zeros((NG, HD), jnp.float32)
        cnt[...] = jnp.zeros((NG, 128), jnp.float32)

    bb = b_ref[...][:, 0]
    oht = (bb[None, :] == lax.broadcasted_iota(jnp.int32, (NG, BN), 0)
           ).astype(jnp.float32)
    sums[...] += jnp.dot(oht, y2_ref[...], preferred_element_type=jnp.float32)
    cnt[...] += jnp.broadcast_to(jnp.sum(oht, axis=1, keepdims=True), (NG, 128))

    @pl.when(i == NBLK - 1)
    def _():
        mean = sums[...] / jnp.maximum(cnt[...][:, 0:1], 1.0)
        out_ref[...] = (jnp.dot(mean, ow_ref[...],
                                preferred_element_type=jnp.float32)
                        + ob_ref[...])


def _pool(y2, batch2d, ow, ob):
    return pl.pallas_call(
        _pool_body,
        grid=(NBLK,),
        in_specs=[pl.BlockSpec((BN, HD), lambda i: (i, 0)),
                  pl.BlockSpec((BN, 1), lambda i: (i, 0)),
                  pl.BlockSpec((HD, OUTD), lambda i: (0, 0)),
                  pl.BlockSpec((1, OUTD), lambda i: (0, 0))],
        out_specs=pl.BlockSpec((NG, OUTD), lambda i: (0, 0)),
        out_shape=jax.ShapeDtypeStruct((NG, OUTD), jnp.float32),
        scratch_shapes=[pltpu.VMEM((NG, HD), jnp.float32),
                        pltpu.VMEM((NG, 128), jnp.float32)],
    )(y2, batch2d, ow, ob)


def kernel(x, edge_index, edge_attr, batch, atom_emb, bond_emb, W1s, b1s,
           bn1_g, bn1_b, W2s, b2s, eps, obn_g, obn_b, out_W, out_b):
    aemb = atom_emb.reshape(9 * 128, HD)
    bemb = bond_emb.reshape(NL, 48, HD)

    h_split = _atom_encode(x, aemb)
    etabs = _etabs(bemb)
    ea_t = edge_attr.T.reshape(3, 1250, 128)
    combo3 = _combo_ids(ea_t).reshape(NSUB, NCHUNK, CHUNK)
    src3 = edge_index[0].reshape(NSUB, NCHUNK, CHUNK)
    dst3 = edge_index[1].reshape(NSUB, NCHUNK, CHUNK)
    idx3 = jnp.stack([src3, dst3, combo3], axis=2)

    for l in range(NL):
        agg_split = _sc_msgpass(h_split, etabs[l], idx3)
        scale = (1.0 + eps[l]).reshape(1, 1)
        y1, st1 = _mlp1(scale, h_split, agg_split, W1s[l],
                        b1s[l].reshape(1, -1))
        y2, st2 = _mlp2(y1, st1, bn1_g[l].reshape(1, -1),
                        bn1_b[l].reshape(1, -1), W2s[l], b2s[l].reshape(1, -1))
        if l < NL - 1:
            h_split = _obn(y2, st2, obn_g[l].reshape(1, -1),
                           obn_b[l].reshape(1, -1))
        else:
            out = _pool(y2, batch.reshape(NN, 1), out_W, out_b.reshape(1, -1))
    return out
